# baseline (speedup 1.0000x reference)
import functools

import jax
import jax.numpy as jnp
from jax import lax
from jax.experimental import pallas as pl
from jax.experimental.pallas import tpu as pltpu
from jax.experimental.pallas import tpu_sc as plsc

B = 16384
M = 1000000
NC = 2
NS = 16
L = 16
NW = NC * NS
SPW = B // NW
NVEC = B // L
RNG = 31264
MPAD = NW * RNG
TAIL = M - (NW - 1) * RNG

D1 = 0.1
D9 = 0.9
MARGIN = 0.2

_mesh = plsc.VectorSubcoreMesh(
    core_axis_name="c", subcore_axis_name="s", num_cores=NC, num_subcores=NS)


def _wid():
    return lax.axis_index("s") * NC + lax.axis_index("c")


def _f32(shape):
    return jax.ShapeDtypeStruct(shape, jnp.float32)


@functools.partial(
    pl.kernel,
    out_type=(
        jax.ShapeDtypeStruct((MPAD,), jnp.int32),
        _f32((B,)), _f32((B,)), _f32((B,)),
        _f32((B,)), _f32((B,)),
    ),
    mesh=_mesh,
    compiler_params=pltpu.CompilerParams(needs_layout_passes=False),
    scratch_types=(
        pltpu.VMEM((B,), jnp.int32),
        pltpu.VMEM((RNG,), jnp.int32),
        pltpu.VMEM((SPW,), jnp.float32),
        pltpu.VMEM((SPW,), jnp.float32),
        pltpu.VMEM((SPW,), jnp.float32),
        pltpu.VMEM((SPW,), jnp.float32),
        pltpu.VMEM((SPW,), jnp.float32),
        pltpu.VMEM((SPW,), jnp.float32),
        pltpu.VMEM((SPW,), jnp.float32),
        pltpu.VMEM((SPW,), jnp.float32),
        pltpu.VMEM((SPW,), jnp.float32),
        pltpu.VMEM((SPW,), jnp.float32),
        pltpu.VMEM((SPW,), jnp.float32),
        pltpu.VMEM((SPW,), jnp.float32),
        pltpu.VMEM((SPW,), jnp.float32),
        pltpu.VMEM((SPW,), jnp.float32),
        pltpu.SemaphoreType.DMA,
    ),
)
def _phase_a(ids, x, y, z, da, db, tg, xst, yst, zst,
             win, updx, updy, updz, evp, los,
             ids_v, win_t, slx, sly, slz, sla, slb, slt,
             gx, gy, gz, bux, buy, buz, bev, blo, sem):
    w = _wid()
    base = w * SPW
    pltpu.sync_copy(ids, ids_v)
    pltpu.sync_copy(x.at[pl.ds(base, SPW)], slx)
    pltpu.sync_copy(y.at[pl.ds(base, SPW)], sly)
    pltpu.sync_copy(z.at[pl.ds(base, SPW)], slz)
    pltpu.sync_copy(da.at[pl.ds(base, SPW)], sla)
    pltpu.sync_copy(db.at[pl.ds(base, SPW)], slb)
    pltpu.sync_copy(tg.at[pl.ds(base, SPW)], slt)
    cps = []
    for c in range(4):
        idxs = ids_v.at[pl.ds(base + c * 128, 128)]
        d = pl.ds(c * 128, 128)
        cps.append(pltpu.async_copy(xst.at[idxs], gx.at[d], sem))
        cps.append(pltpu.async_copy(yst.at[idxs], gy.at[d], sem))
        cps.append(pltpu.async_copy(zst.at[idxs], gz.at[d], sem))

    _winner_scan(ids_v, win_t, w)
    pltpu.sync_copy(win_t, win.at[pl.ds(w * RNG, RNG)])
    for cp in cps:
        cp.wait()

    def cbody(j, carry):
        s = pl.ds(j * L, L)
        ex = jnp.exp(slx[s])
        ey = jnp.exp(sly[s])
        ez = jnp.exp(slz[s])
        bux[s] = D1 * ex + D9 * gx[s]
        buy[s] = D1 * ey + D9 * gy[s]
        buz[s] = D1 * ez + D9 * gz[s]
        bev[s] = ex * ey * ez
        blo[s] = jnp.maximum(0.0, -slt[s] * (sla[s] - slb[s]) + MARGIN)
        return carry

    lax.fori_loop(0, SPW // L, cbody, 0)
    pltpu.sync_copy(bux, updx.at[pl.ds(base, SPW)])
    pltpu.sync_copy(buy, updy.at[pl.ds(base, SPW)])
    pltpu.sync_copy(buz, updz.at[pl.ds(base, SPW)])
    pltpu.sync_copy(bev, evp.at[pl.ds(base, SPW)])
    pltpu.sync_copy(blo, los.at[pl.ds(base, SPW)])


def _winner_scan(ids_v, win_t, w):
    rbase = w * RNG
    iota = lax.iota(jnp.int32, L)

    G = 8

    def wbody(g, carry):
        offs_l, m_l, val_l = [], [], []
        for t in range(G):
            v = g * G + t
            idv = ids_v[pl.ds(v * L, L)]
            off = idv - rbase
            m = (off >= 0) & (off < RNG)
            offs = jnp.where(m, off, 0)
            val = v * L + iota
            plsc.store_scatter(win_t, [offs], val, mask=m)
            offs_l.append(offs)
            m_l.append(m)
            val_l.append(val)
        bad = None
        for t in range(G):
            got = plsc.load_gather(win_t, [offs_l[t]])
            bt = m_l[t] & (got != val_l[t])
            bad = bt if bad is None else (bad | bt)

        @pl.when(jnp.any(bad))
        def _fix():
            for t in range(G):
                for lane in range(L):
                    plsc.store_scatter(win_t, [offs_l[t]], val_l[t],
                                       mask=m_l[t] & (iota == lane))

        return carry

    lax.fori_loop(0, NVEC // G, wbody, 0)


@functools.partial(
    pl.kernel,
    out_type=(
        _f32((B,)),
        _f32((NW, L)),
        _f32((NW, L)),
    ),
    mesh=_mesh,
    compiler_params=pltpu.CompilerParams(needs_layout_passes=False),
    scratch_types=(
        pltpu.VMEM((4, 128), jnp.int32),
        pltpu.VMEM((4, 128), jnp.int32),
        pltpu.VMEM((SPW,), jnp.float32),
        pltpu.VMEM((SPW,), jnp.float32),
        pltpu.VMEM((SPW,), jnp.float32),
        pltpu.VMEM((SPW,), jnp.float32),
        pltpu.VMEM((SPW,), jnp.float32),
        pltpu.VMEM((SPW,), jnp.float32),
        pltpu.VMEM((SPW,), jnp.float32),
        pltpu.VMEM((SPW,), jnp.float32),
        pltpu.VMEM((SPW,), jnp.float32),
        pltpu.VMEM((SPW,), jnp.float32),
        pltpu.VMEM((SPW,), jnp.int32),
        pltpu.VMEM((SPW,), jnp.float32),
        pltpu.VMEM((SPW,), jnp.float32),
        pltpu.VMEM((L,), jnp.float32),
        pltpu.VMEM((L,), jnp.float32),
        pltpu.SemaphoreType.DMA,
    ),
)
def _phase_b(ids, evph, losh, updx, updy, updz, winh, sval, sw, oval, ow,
             wout, Sp, Swp,
             ids2d, win2d, sle, sll, kx, ky, kz, ew, lw, gv, gw,
             bw, ids1d, bk2, bk2w, pS, pSw, sem):
    w_ = _wid()
    base = w_ * SPW
    for c in range(4):
        pltpu.sync_copy(ids.at[pl.ds(base + c * 128, 128)], ids2d.at[c])
    pltpu.sync_copy(ids.at[pl.ds(base, SPW)], ids1d)
    pltpu.sync_copy(evph.at[pl.ds(base, SPW)], sle)
    pltpu.sync_copy(losh.at[pl.ds(base, SPW)], sll)
    cps = [pltpu.async_copy(winh.at[ids2d.at[c]], win2d.at[c], sem)
           for c in range(4)]
    for cp in cps:
        cp.wait()
    cps = []
    for c in range(4):
        iw = win2d.at[c]
        ii = ids2d.at[c]
        d = pl.ds(c * 128, 128)
        cps.append(pltpu.async_copy(updx.at[iw], kx.at[d], sem))
        cps.append(pltpu.async_copy(updy.at[iw], ky.at[d], sem))
        cps.append(pltpu.async_copy(updz.at[iw], kz.at[d], sem))
        cps.append(pltpu.async_copy(evph.at[iw], ew.at[d], sem))
        cps.append(pltpu.async_copy(losh.at[iw], lw.at[d], sem))
        cps.append(pltpu.async_copy(sval.at[ii], gv.at[d], sem))
        cps.append(pltpu.async_copy(sw.at[ii], gw.at[d], sem))
    for cp in cps:
        cp.wait()

    def cbody(j, carry):
        sS, sSw = carry
        s = pl.ds(j * L, L)
        kprod = kx[s] * ky[s] * kz[s]
        wv = sle[s] / kprod
        wwin = ew[s] / kprod
        gws = gw[s]
        nww = D1 * wwin + D9 * gws
        nvv = (D1 * wwin * lw[s] + D9 * gws * gv[s]) / nww
        bw[s] = wv
        bk2[s] = nvv
        bk2w[s] = nww
        return sS + (sll[s] - nvv) * wv, sSw + wv

    zero = jnp.zeros((L,), jnp.float32)
    sS, sSw = lax.fori_loop(0, SPW // L, cbody, (zero, zero))
    pS[...] = sS
    pSw[...] = sSw
    cps = [pltpu.async_copy(bk2, oval.at[ids1d], sem),
           pltpu.async_copy(bk2w, ow.at[ids1d], sem)]
    pltpu.sync_copy(bw, wout.at[pl.ds(base, SPW)])
    pltpu.sync_copy(pS, Sp.at[w_])
    pltpu.sync_copy(pSw, Swp.at[w_])
    for cp in cps:
        cp.wait()


def kernel(dist_a, dist_b, x, y, z, target, ids, xstore, ystore, zstore,
           store_val, store_w):
    win, updx, updy, updz, evp, los = _phase_a(
        ids, x, y, z, dist_a, dist_b, target, xstore, ystore, zstore)
    oval_ref = jax.new_ref(store_val)
    ow_ref = jax.new_ref(store_w)
    wout, Sp, Swp = _phase_b(
        ids, evp, los, updx, updy, updz, win, store_val, store_w,
        oval_ref, ow_ref)
    n = (Swp.sum() + 1e-5) / B
    final = Sp.sum() / n
    return final, wout, oval_ref[...], ow_ref[...]


# speedup vs baseline: 1.1272x; 1.1272x over previous
import functools

import jax
import jax.numpy as jnp
from jax import lax
from jax.experimental import pallas as pl
from jax.experimental.pallas import tpu as pltpu
from jax.experimental.pallas import tpu_sc as plsc

B = 16384
M = 1000000
NC = 2
NS = 16
L = 16
NW = NC * NS
SPW = B // NW
NVEC = B // L
RNG = 31264
MPAD = NW * RNG
TAIL = M - (NW - 1) * RNG

D1 = 0.1
D9 = 0.9
MARGIN = 0.2

_mesh = plsc.VectorSubcoreMesh(
    core_axis_name="c", subcore_axis_name="s", num_cores=NC, num_subcores=NS)


def _wid():
    return lax.axis_index("s") * NC + lax.axis_index("c")


def _f32(shape):
    return jax.ShapeDtypeStruct(shape, jnp.float32)


@functools.partial(
    pl.kernel,
    out_type=(
        jax.ShapeDtypeStruct((MPAD,), jnp.int32),
        _f32((B,)), _f32((B,)), _f32((B,)),
        _f32((B,)), _f32((B,)),
    ),
    mesh=_mesh,
    compiler_params=pltpu.CompilerParams(needs_layout_passes=False),
    scratch_types=(
        pltpu.VMEM((B,), jnp.int32),
        pltpu.VMEM((RNG,), jnp.int32),
        pltpu.VMEM((SPW,), jnp.float32),
        pltpu.VMEM((SPW,), jnp.float32),
        pltpu.VMEM((SPW,), jnp.float32),
        pltpu.VMEM((SPW,), jnp.float32),
        pltpu.VMEM((SPW,), jnp.float32),
        pltpu.VMEM((SPW,), jnp.float32),
        pltpu.VMEM((SPW,), jnp.float32),
        pltpu.VMEM((SPW,), jnp.float32),
        pltpu.VMEM((SPW,), jnp.float32),
        pltpu.VMEM((SPW,), jnp.float32),
        pltpu.VMEM((SPW,), jnp.float32),
        pltpu.VMEM((SPW,), jnp.float32),
        pltpu.VMEM((SPW,), jnp.float32),
        pltpu.VMEM((SPW,), jnp.float32),
        pltpu.SemaphoreType.DMA,
    ),
)
def _phase_a(ids, x, y, z, da, db, tg, xst, yst, zst,
             win, updx, updy, updz, evp, los,
             ids_v, win_t, slx, sly, slz, sla, slb, slt,
             gx, gy, gz, bux, buy, buz, bev, blo, sem):
    w = _wid()
    base = w * SPW
    pltpu.sync_copy(ids, ids_v)
    pltpu.sync_copy(x.at[pl.ds(base, SPW)], slx)
    pltpu.sync_copy(y.at[pl.ds(base, SPW)], sly)
    pltpu.sync_copy(z.at[pl.ds(base, SPW)], slz)
    pltpu.sync_copy(da.at[pl.ds(base, SPW)], sla)
    pltpu.sync_copy(db.at[pl.ds(base, SPW)], slb)
    pltpu.sync_copy(tg.at[pl.ds(base, SPW)], slt)
    cps = []
    for c in range(4):
        idxs = ids_v.at[pl.ds(base + c * 128, 128)]
        d = pl.ds(c * 128, 128)
        cps.append(pltpu.async_copy(xst.at[idxs], gx.at[d], sem))
        cps.append(pltpu.async_copy(yst.at[idxs], gy.at[d], sem))
        cps.append(pltpu.async_copy(zst.at[idxs], gz.at[d], sem))

    _winner_scan(ids_v, win_t, w)
    pltpu.sync_copy(win_t, win.at[pl.ds(w * RNG, RNG)])
    for cp in cps:
        cp.wait()

    def cbody(j, carry):
        s = pl.ds(j * L, L)
        ex = jnp.exp(slx[s])
        ey = jnp.exp(sly[s])
        ez = jnp.exp(slz[s])
        bux[s] = D1 * ex + D9 * gx[s]
        buy[s] = D1 * ey + D9 * gy[s]
        buz[s] = D1 * ez + D9 * gz[s]
        bev[s] = ex * ey * ez
        blo[s] = jnp.maximum(0.0, -slt[s] * (sla[s] - slb[s]) + MARGIN)
        return carry

    lax.fori_loop(0, SPW // L, cbody, 0)
    pltpu.sync_copy(bux, updx.at[pl.ds(base, SPW)])
    pltpu.sync_copy(buy, updy.at[pl.ds(base, SPW)])
    pltpu.sync_copy(buz, updz.at[pl.ds(base, SPW)])
    pltpu.sync_copy(bev, evp.at[pl.ds(base, SPW)])
    pltpu.sync_copy(blo, los.at[pl.ds(base, SPW)])


def _winner_scan(ids_v, win_t, w):
    rbase = w * RNG
    iota = lax.iota(jnp.int32, L)

    G = 8

    def wbody(g, carry):
        offs_l, m_l, val_l = [], [], []
        for t in range(G):
            v = g * G + t
            idv = ids_v[pl.ds(v * L, L)]
            off = idv - rbase
            m = (off >= 0) & (off < RNG)
            offs = jnp.where(m, off, 0)
            val = v * L + iota
            plsc.store_scatter(win_t, [offs], val, mask=m)
            offs_l.append(offs)
            m_l.append(m)
            val_l.append(val)
        bad = None
        for t in range(G):
            got = plsc.load_gather(win_t, [offs_l[t]])
            bt = m_l[t] & (got != val_l[t])
            bad = bt if bad is None else (bad | bt)

        @pl.when(jnp.any(bad))
        def _fix():
            for t in range(G):
                for lane in range(L):
                    plsc.store_scatter(win_t, [offs_l[t]], val_l[t],
                                       mask=m_l[t] & (iota == lane))

        return carry

    lax.fori_loop(0, NVEC // G, wbody, 0)


@functools.partial(
    pl.kernel,
    out_type=(
        _f32((B,)),
        _f32((B,)),
        _f32((B,)),
        _f32((NW, L)),
        _f32((NW, L)),
    ),
    mesh=_mesh,
    compiler_params=pltpu.CompilerParams(needs_layout_passes=False),
    scratch_types=(
        pltpu.VMEM((4, 128), jnp.int32),
        pltpu.VMEM((4, 128), jnp.int32),
        pltpu.VMEM((SPW,), jnp.float32),
        pltpu.VMEM((SPW,), jnp.float32),
        pltpu.VMEM((SPW,), jnp.float32),
        pltpu.VMEM((SPW,), jnp.float32),
        pltpu.VMEM((SPW,), jnp.float32),
        pltpu.VMEM((SPW,), jnp.float32),
        pltpu.VMEM((SPW,), jnp.float32),
        pltpu.VMEM((SPW,), jnp.float32),
        pltpu.VMEM((SPW,), jnp.float32),
        pltpu.VMEM((SPW,), jnp.float32),
        pltpu.VMEM((SPW,), jnp.float32),
        pltpu.VMEM((SPW,), jnp.float32),
        pltpu.VMEM((L,), jnp.float32),
        pltpu.VMEM((L,), jnp.float32),
        pltpu.SemaphoreType.DMA,
    ),
)
def _phase_b(ids, evph, losh, updx, updy, updz, winh, sval, sw,
             wout, k2o, k2wo, Sp, Swp,
             ids2d, win2d, sle, sll, kx, ky, kz, ew, lw, gv, gw,
             bw, bk2, bk2w, pS, pSw, sem):
    w_ = _wid()
    base = w_ * SPW
    for c in range(4):
        pltpu.sync_copy(ids.at[pl.ds(base + c * 128, 128)], ids2d.at[c])
    pltpu.sync_copy(evph.at[pl.ds(base, SPW)], sle)
    pltpu.sync_copy(losh.at[pl.ds(base, SPW)], sll)
    cps = [pltpu.async_copy(winh.at[ids2d.at[c]], win2d.at[c], sem)
           for c in range(4)]
    for cp in cps:
        cp.wait()
    cps = []
    for c in range(4):
        iw = win2d.at[c]
        ii = ids2d.at[c]
        d = pl.ds(c * 128, 128)
        cps.append(pltpu.async_copy(updx.at[iw], kx.at[d], sem))
        cps.append(pltpu.async_copy(updy.at[iw], ky.at[d], sem))
        cps.append(pltpu.async_copy(updz.at[iw], kz.at[d], sem))
        cps.append(pltpu.async_copy(evph.at[iw], ew.at[d], sem))
        cps.append(pltpu.async_copy(losh.at[iw], lw.at[d], sem))
        cps.append(pltpu.async_copy(sval.at[ii], gv.at[d], sem))
        cps.append(pltpu.async_copy(sw.at[ii], gw.at[d], sem))
    for cp in cps:
        cp.wait()

    def cbody(j, carry):
        sS, sSw = carry
        s = pl.ds(j * L, L)
        kprod = kx[s] * ky[s] * kz[s]
        wv = sle[s] / kprod
        wwin = ew[s] / kprod
        gws = gw[s]
        nww = D1 * wwin + D9 * gws
        nvv = (D1 * wwin * lw[s] + D9 * gws * gv[s]) / nww
        bw[s] = wv
        bk2[s] = nvv
        bk2w[s] = nww
        return sS + (sll[s] - nvv) * wv, sSw + wv

    zero = jnp.zeros((L,), jnp.float32)
    sS, sSw = lax.fori_loop(0, SPW // L, cbody, (zero, zero))
    pS[...] = sS
    pSw[...] = sSw
    pltpu.sync_copy(bw, wout.at[pl.ds(base, SPW)])
    pltpu.sync_copy(bk2, k2o.at[pl.ds(base, SPW)])
    pltpu.sync_copy(bk2w, k2wo.at[pl.ds(base, SPW)])
    pltpu.sync_copy(pS, Sp.at[w_])
    pltpu.sync_copy(pSw, Swp.at[w_])


@functools.partial(
    pl.kernel,
    out_type=(_f32((M,)), _f32((M,))),
    mesh=_mesh,
    compiler_params=pltpu.CompilerParams(needs_layout_passes=False),
    scratch_types=(
        pltpu.VMEM((B,), jnp.int32),
        pltpu.VMEM((B,), jnp.float32),
        pltpu.VMEM((B,), jnp.float32),
        pltpu.VMEM((RNG,), jnp.float32),
        pltpu.VMEM((RNG,), jnp.float32),
        pltpu.SemaphoreType.DMA,
    ),
)
def _phase_c(ids, k2, k2w, sval, sw,
             oval, ow,
             ids_v, k2v, k2wv, segv, segw, sem):
    w_ = _wid()
    rbase = w_ * RNG
    last = NW - 1
    cps = [pltpu.async_copy(ids, ids_v, sem),
           pltpu.async_copy(k2, k2v, sem),
           pltpu.async_copy(k2w, k2wv, sem)]

    @pl.when(w_ < last)
    def _load_full():
        a = pltpu.async_copy(sval.at[pl.ds(rbase, RNG)], segv, sem)
        b = pltpu.async_copy(sw.at[pl.ds(rbase, RNG)], segw, sem)
        a.wait()
        b.wait()

    @pl.when(w_ == last)
    def _load_tail():
        a = pltpu.async_copy(sval.at[pl.ds(rbase, TAIL)],
                             segv.at[pl.ds(0, TAIL)], sem)
        b = pltpu.async_copy(sw.at[pl.ds(rbase, TAIL)],
                             segw.at[pl.ds(0, TAIL)], sem)
        a.wait()
        b.wait()

    for cp in cps:
        cp.wait()

    def abody(v, carry):
        s = pl.ds(v * L, L)
        idv = ids_v[s]
        off = idv - rbase
        m = (off >= 0) & (off < RNG)
        offs = jnp.where(m, off, 0)
        plsc.store_scatter(segv, [offs], k2v[s], mask=m)
        plsc.store_scatter(segw, [offs], k2wv[s], mask=m)
        return carry

    lax.fori_loop(0, NVEC, abody, 0, unroll=8)

    @pl.when(w_ < last)
    def _store_full():
        pltpu.sync_copy(segv, oval.at[pl.ds(rbase, RNG)])
        pltpu.sync_copy(segw, ow.at[pl.ds(rbase, RNG)])

    @pl.when(w_ == last)
    def _store_tail():
        pltpu.sync_copy(segv.at[pl.ds(0, TAIL)], oval.at[pl.ds(rbase, TAIL)])
        pltpu.sync_copy(segw.at[pl.ds(0, TAIL)], ow.at[pl.ds(rbase, TAIL)])


def kernel(dist_a, dist_b, x, y, z, target, ids, xstore, ystore, zstore,
           store_val, store_w):
    win, updx, updy, updz, evp, los = _phase_a(
        ids, x, y, z, dist_a, dist_b, target, xstore, ystore, zstore)
    wout, k2, k2w, Sp, Swp = _phase_b(
        ids, evp, los, updx, updy, updz, win, store_val, store_w)
    oval, ow = _phase_c(ids, k2, k2w, store_val, store_w)
    n = (Swp.sum() + 1e-5) / B
    final = Sp.sum() / n
    return final, wout, oval, ow


# speedup vs baseline: 1.1594x; 1.0286x over previous
import functools

import jax
import jax.numpy as jnp
from jax import lax
from jax.experimental import pallas as pl
from jax.experimental.pallas import tpu as pltpu
from jax.experimental.pallas import tpu_sc as plsc

B = 16384
M = 1000000
NC = 2
NS = 16
L = 16
NW = NC * NS
SPW = B // NW
NVEC = B // L
RNG = 31264
MPAD = NW * RNG
TAIL = M - (NW - 1) * RNG

D1 = 0.1
D9 = 0.9
MARGIN = 0.2

_mesh = plsc.VectorSubcoreMesh(
    core_axis_name="c", subcore_axis_name="s", num_cores=NC, num_subcores=NS)


def _wid():
    return lax.axis_index("s") * NC + lax.axis_index("c")


def _f32(shape):
    return jax.ShapeDtypeStruct(shape, jnp.float32)


@functools.partial(
    pl.kernel,
    out_type=(
        jax.ShapeDtypeStruct((MPAD,), jnp.int32),
        _f32((B,)), _f32((B,)), _f32((B,)),
        _f32((B,)), _f32((B,)),
    ),
    mesh=_mesh,
    compiler_params=pltpu.CompilerParams(needs_layout_passes=False),
    scratch_types=(
        pltpu.VMEM((B,), jnp.int32),
        pltpu.VMEM((RNG,), jnp.int32),
        pltpu.VMEM((SPW,), jnp.float32),
        pltpu.VMEM((SPW,), jnp.float32),
        pltpu.VMEM((SPW,), jnp.float32),
        pltpu.VMEM((SPW,), jnp.float32),
        pltpu.VMEM((SPW,), jnp.float32),
        pltpu.VMEM((SPW,), jnp.float32),
        pltpu.VMEM((SPW,), jnp.float32),
        pltpu.VMEM((SPW,), jnp.float32),
        pltpu.VMEM((SPW,), jnp.float32),
        pltpu.VMEM((SPW,), jnp.float32),
        pltpu.VMEM((SPW,), jnp.float32),
        pltpu.VMEM((SPW,), jnp.float32),
        pltpu.VMEM((SPW,), jnp.float32),
        pltpu.VMEM((SPW,), jnp.float32),
        pltpu.SemaphoreType.DMA,
    ),
)
def _phase_a(ids, x, y, z, da, db, tg, xst, yst, zst,
             win, updx, updy, updz, evp, los,
             ids_v, win_t, slx, sly, slz, sla, slb, slt,
             gx, gy, gz, bux, buy, buz, bev, blo, sem):
    w = _wid()
    base = w * SPW
    pltpu.sync_copy(ids, ids_v)
    pltpu.sync_copy(x.at[pl.ds(base, SPW)], slx)
    pltpu.sync_copy(y.at[pl.ds(base, SPW)], sly)
    pltpu.sync_copy(z.at[pl.ds(base, SPW)], slz)
    pltpu.sync_copy(da.at[pl.ds(base, SPW)], sla)
    pltpu.sync_copy(db.at[pl.ds(base, SPW)], slb)
    pltpu.sync_copy(tg.at[pl.ds(base, SPW)], slt)
    cps = []
    for c in range(4):
        idxs = ids_v.at[pl.ds(base + c * 128, 128)]
        d = pl.ds(c * 128, 128)
        cps.append(pltpu.async_copy(xst.at[idxs], gx.at[d], sem))
        cps.append(pltpu.async_copy(yst.at[idxs], gy.at[d], sem))
        cps.append(pltpu.async_copy(zst.at[idxs], gz.at[d], sem))

    _winner_scan(ids_v, win_t, w)
    pltpu.sync_copy(win_t, win.at[pl.ds(w * RNG, RNG)])
    for cp in cps:
        cp.wait()

    def cbody(j, carry):
        s = pl.ds(j * L, L)
        ex = jnp.exp(slx[s])
        ey = jnp.exp(sly[s])
        ez = jnp.exp(slz[s])
        bux[s] = D1 * ex + D9 * gx[s]
        buy[s] = D1 * ey + D9 * gy[s]
        buz[s] = D1 * ez + D9 * gz[s]
        bev[s] = ex * ey * ez
        blo[s] = jnp.maximum(0.0, -slt[s] * (sla[s] - slb[s]) + MARGIN)
        return carry

    lax.fori_loop(0, SPW // L, cbody, 0)
    pltpu.sync_copy(bux, updx.at[pl.ds(base, SPW)])
    pltpu.sync_copy(buy, updy.at[pl.ds(base, SPW)])
    pltpu.sync_copy(buz, updz.at[pl.ds(base, SPW)])
    pltpu.sync_copy(bev, evp.at[pl.ds(base, SPW)])
    pltpu.sync_copy(blo, los.at[pl.ds(base, SPW)])


def _winner_scan(ids_v, win_t, w):
    rbase = w * RNG
    iota = lax.iota(jnp.int32, L)

    G = 16

    def wbody(g, carry):
        offs_l, m_l, val_l = [], [], []
        for t in range(G):
            v = g * G + t
            idv = ids_v[pl.ds(v * L, L)]
            off = idv - rbase
            m = (off >= 0) & (off < RNG)
            offs = jnp.where(m, off, 0)
            val = v * L + iota
            plsc.store_scatter(win_t, [offs], val, mask=m)
            offs_l.append(offs)
            m_l.append(m)
            val_l.append(val)
        bad = None
        for t in range(G):
            got = plsc.load_gather(win_t, [offs_l[t]])
            bt = m_l[t] & (got != val_l[t])
            bad = bt if bad is None else (bad | bt)

        @pl.when(jnp.any(bad))
        def _fix():
            for t in range(G):
                for lane in range(L):
                    plsc.store_scatter(win_t, [offs_l[t]], val_l[t],
                                       mask=m_l[t] & (iota == lane))

        return carry

    lax.fori_loop(0, NVEC // G, wbody, 0)


@functools.partial(
    pl.kernel,
    out_type=(
        _f32((B,)),
        _f32((B,)),
        _f32((B,)),
        _f32((NW, L)),
        _f32((NW, L)),
    ),
    mesh=_mesh,
    compiler_params=pltpu.CompilerParams(needs_layout_passes=False),
    scratch_types=(
        pltpu.VMEM((4, 128), jnp.int32),
        pltpu.VMEM((4, 128), jnp.int32),
        pltpu.VMEM((SPW,), jnp.float32),
        pltpu.VMEM((SPW,), jnp.float32),
        pltpu.VMEM((SPW,), jnp.float32),
        pltpu.VMEM((SPW,), jnp.float32),
        pltpu.VMEM((SPW,), jnp.float32),
        pltpu.VMEM((SPW,), jnp.float32),
        pltpu.VMEM((SPW,), jnp.float32),
        pltpu.VMEM((SPW,), jnp.float32),
        pltpu.VMEM((SPW,), jnp.float32),
        pltpu.VMEM((SPW,), jnp.float32),
        pltpu.VMEM((SPW,), jnp.float32),
        pltpu.VMEM((SPW,), jnp.float32),
        pltpu.VMEM((L,), jnp.float32),
        pltpu.VMEM((L,), jnp.float32),
        pltpu.SemaphoreType.DMA,
    ),
)
def _phase_b(ids, evph, losh, updx, updy, updz, winh, sval, sw,
             wout, k2o, k2wo, Sp, Swp,
             ids2d, win2d, sle, sll, kx, ky, kz, ew, lw, gv, gw,
             bw, bk2, bk2w, pS, pSw, sem):
    w_ = _wid()
    base = w_ * SPW
    cps = [pltpu.async_copy(ids.at[pl.ds(base + c * 128, 128)], ids2d.at[c],
                            sem) for c in range(4)]
    cps.append(pltpu.async_copy(evph.at[pl.ds(base, SPW)], sle, sem))
    cps.append(pltpu.async_copy(losh.at[pl.ds(base, SPW)], sll, sem))
    for cp in cps:
        cp.wait()
    cps = [pltpu.async_copy(winh.at[ids2d.at[c]], win2d.at[c], sem)
           for c in range(4)]
    for cp in cps:
        cp.wait()
    cps = []
    for c in range(4):
        iw = win2d.at[c]
        ii = ids2d.at[c]
        d = pl.ds(c * 128, 128)
        cps.append(pltpu.async_copy(updx.at[iw], kx.at[d], sem))
        cps.append(pltpu.async_copy(updy.at[iw], ky.at[d], sem))
        cps.append(pltpu.async_copy(updz.at[iw], kz.at[d], sem))
        cps.append(pltpu.async_copy(evph.at[iw], ew.at[d], sem))
        cps.append(pltpu.async_copy(losh.at[iw], lw.at[d], sem))
        cps.append(pltpu.async_copy(sval.at[ii], gv.at[d], sem))
        cps.append(pltpu.async_copy(sw.at[ii], gw.at[d], sem))
    for cp in cps:
        cp.wait()

    def cbody(j, carry):
        sS, sSw = carry
        s = pl.ds(j * L, L)
        kprod = kx[s] * ky[s] * kz[s]
        wv = sle[s] / kprod
        wwin = ew[s] / kprod
        gws = gw[s]
        nww = D1 * wwin + D9 * gws
        nvv = (D1 * wwin * lw[s] + D9 * gws * gv[s]) / nww
        bw[s] = wv
        bk2[s] = nvv
        bk2w[s] = nww
        return sS + (sll[s] - nvv) * wv, sSw + wv

    zero = jnp.zeros((L,), jnp.float32)
    sS, sSw = lax.fori_loop(0, SPW // L, cbody, (zero, zero))
    pS[...] = sS
    pSw[...] = sSw
    pltpu.sync_copy(bw, wout.at[pl.ds(base, SPW)])
    pltpu.sync_copy(bk2, k2o.at[pl.ds(base, SPW)])
    pltpu.sync_copy(bk2w, k2wo.at[pl.ds(base, SPW)])
    pltpu.sync_copy(pS, Sp.at[w_])
    pltpu.sync_copy(pSw, Swp.at[w_])


@functools.partial(
    pl.kernel,
    out_type=(_f32((M,)), _f32((M,))),
    mesh=_mesh,
    compiler_params=pltpu.CompilerParams(needs_layout_passes=False),
    scratch_types=(
        pltpu.VMEM((B,), jnp.int32),
        pltpu.VMEM((B,), jnp.float32),
        pltpu.VMEM((B,), jnp.float32),
        pltpu.VMEM((RNG,), jnp.float32),
        pltpu.VMEM((RNG,), jnp.float32),
        pltpu.SemaphoreType.DMA,
    ),
)
def _phase_c(ids, k2, k2w, sval, sw,
             oval, ow,
             ids_v, k2v, k2wv, segv, segw, sem):
    w_ = _wid()
    rbase = w_ * RNG
    last = NW - 1
    cps = [pltpu.async_copy(ids, ids_v, sem),
           pltpu.async_copy(k2, k2v, sem),
           pltpu.async_copy(k2w, k2wv, sem)]

    @pl.when(w_ < last)
    def _load_full():
        a = pltpu.async_copy(sval.at[pl.ds(rbase, RNG)], segv, sem)
        b = pltpu.async_copy(sw.at[pl.ds(rbase, RNG)], segw, sem)
        a.wait()
        b.wait()

    @pl.when(w_ == last)
    def _load_tail():
        a = pltpu.async_copy(sval.at[pl.ds(rbase, TAIL)],
                             segv.at[pl.ds(0, TAIL)], sem)
        b = pltpu.async_copy(sw.at[pl.ds(rbase, TAIL)],
                             segw.at[pl.ds(0, TAIL)], sem)
        a.wait()
        b.wait()

    for cp in cps:
        cp.wait()

    def abody(v, carry):
        s = pl.ds(v * L, L)
        idv = ids_v[s]
        off = idv - rbase
        m = (off >= 0) & (off < RNG)
        offs = jnp.where(m, off, 0)
        plsc.store_scatter(segv, [offs], k2v[s], mask=m)
        plsc.store_scatter(segw, [offs], k2wv[s], mask=m)
        return carry

    lax.fori_loop(0, NVEC, abody, 0, unroll=8)

    @pl.when(w_ < last)
    def _store_full():
        a = pltpu.async_copy(segv, oval.at[pl.ds(rbase, RNG)], sem)
        b = pltpu.async_copy(segw, ow.at[pl.ds(rbase, RNG)], sem)
        a.wait()
        b.wait()

    @pl.when(w_ == last)
    def _store_tail():
        a = pltpu.async_copy(segv.at[pl.ds(0, TAIL)],
                             oval.at[pl.ds(rbase, TAIL)], sem)
        b = pltpu.async_copy(segw.at[pl.ds(0, TAIL)],
                             ow.at[pl.ds(rbase, TAIL)], sem)
        a.wait()
        b.wait()


def kernel(dist_a, dist_b, x, y, z, target, ids, xstore, ystore, zstore,
           store_val, store_w):
    win, updx, updy, updz, evp, los = _phase_a(
        ids, x, y, z, dist_a, dist_b, target, xstore, ystore, zstore)
    wout, k2, k2w, Sp, Swp = _phase_b(
        ids, evp, los, updx, updy, updz, win, store_val, store_w)
    oval, ow = _phase_c(ids, k2, k2w, store_val, store_w)
    n = (Swp.sum() + 1e-5) / B
    final = Sp.sum() / n
    return final, wout, oval, ow


# speedup vs baseline: 1.2191x; 1.0515x over previous
import functools

import jax
import jax.numpy as jnp
from jax import lax
from jax.experimental import pallas as pl
from jax.experimental.pallas import tpu as pltpu
from jax.experimental.pallas import tpu_sc as plsc

B = 16384
M = 1000000
NC = 2
NS = 16
L = 16
NW = NC * NS
SPW = B // NW
NVEC = B // L
RNG = 31264
MPAD = NW * RNG
TAIL = M - (NW - 1) * RNG

D1 = 0.1
D9 = 0.9
MARGIN = 0.2

_mesh = plsc.VectorSubcoreMesh(
    core_axis_name="c", subcore_axis_name="s", num_cores=NC, num_subcores=NS)


def _wid():
    return lax.axis_index("s") * NC + lax.axis_index("c")


def _f32(shape):
    return jax.ShapeDtypeStruct(shape, jnp.float32)


@functools.partial(
    pl.kernel,
    out_type=(
        jax.ShapeDtypeStruct((MPAD,), jnp.int32),
        _f32((B,)), _f32((B,)), _f32((B,)),
        _f32((B,)), _f32((B,)),
    ),
    mesh=_mesh,
    compiler_params=pltpu.CompilerParams(needs_layout_passes=False),
    scratch_types=(
        pltpu.VMEM((B,), jnp.int32),
        pltpu.VMEM((RNG,), jnp.int32),
        pltpu.VMEM((SPW,), jnp.float32),
        pltpu.VMEM((SPW,), jnp.float32),
        pltpu.VMEM((SPW,), jnp.float32),
        pltpu.VMEM((SPW,), jnp.float32),
        pltpu.VMEM((SPW,), jnp.float32),
        pltpu.VMEM((SPW,), jnp.float32),
        pltpu.VMEM((SPW,), jnp.float32),
        pltpu.VMEM((SPW,), jnp.float32),
        pltpu.VMEM((SPW,), jnp.float32),
        pltpu.VMEM((SPW,), jnp.float32),
        pltpu.VMEM((SPW,), jnp.float32),
        pltpu.VMEM((SPW,), jnp.float32),
        pltpu.VMEM((SPW,), jnp.float32),
        pltpu.VMEM((SPW,), jnp.float32),
        pltpu.SemaphoreType.DMA,
    ),
)
def _phase_a(ids, x, y, z, da, db, tg, xst, yst, zst,
             win, updx, updy, updz, evp, los,
             ids_v, win_t, slx, sly, slz, sla, slb, slt,
             gx, gy, gz, bux, buy, buz, bev, blo, sem):
    w = _wid()
    base = w * SPW
    slcps = [pltpu.async_copy(x.at[pl.ds(base, SPW)], slx, sem),
             pltpu.async_copy(y.at[pl.ds(base, SPW)], sly, sem),
             pltpu.async_copy(z.at[pl.ds(base, SPW)], slz, sem),
             pltpu.async_copy(da.at[pl.ds(base, SPW)], sla, sem),
             pltpu.async_copy(db.at[pl.ds(base, SPW)], slb, sem),
             pltpu.async_copy(tg.at[pl.ds(base, SPW)], slt, sem)]
    pltpu.sync_copy(ids, ids_v)
    cps = []
    for c in range(4):
        idxs = ids_v.at[pl.ds(base + c * 128, 128)]
        d = pl.ds(c * 128, 128)
        cps.append(pltpu.async_copy(xst.at[idxs], gx.at[d], sem))
        cps.append(pltpu.async_copy(yst.at[idxs], gy.at[d], sem))
        cps.append(pltpu.async_copy(zst.at[idxs], gz.at[d], sem))

    _winner_scan(ids_v, win_t, w)
    wcp = pltpu.async_copy(win_t, win.at[pl.ds(w * RNG, RNG)], sem)
    for cp in slcps:
        cp.wait()
    for cp in cps:
        cp.wait()

    def cbody(j, carry):
        s = pl.ds(j * L, L)
        ex = jnp.exp(slx[s])
        ey = jnp.exp(sly[s])
        ez = jnp.exp(slz[s])
        bux[s] = D1 * ex + D9 * gx[s]
        buy[s] = D1 * ey + D9 * gy[s]
        buz[s] = D1 * ez + D9 * gz[s]
        bev[s] = ex * ey * ez
        blo[s] = jnp.maximum(0.0, -slt[s] * (sla[s] - slb[s]) + MARGIN)
        return carry

    lax.fori_loop(0, SPW // L, cbody, 0)
    pcps = [pltpu.async_copy(bux, updx.at[pl.ds(base, SPW)], sem),
            pltpu.async_copy(buy, updy.at[pl.ds(base, SPW)], sem),
            pltpu.async_copy(buz, updz.at[pl.ds(base, SPW)], sem),
            pltpu.async_copy(bev, evp.at[pl.ds(base, SPW)], sem),
            pltpu.async_copy(blo, los.at[pl.ds(base, SPW)], sem)]
    wcp.wait()
    for cp in pcps:
        cp.wait()


def _winner_scan(ids_v, win_t, w):
    rbase = w * RNG
    iota = lax.iota(jnp.int32, L)

    G = 16

    def wbody(g, carry):
        offs_l, m_l, val_l = [], [], []
        for t in range(G):
            v = g * G + t
            idv = ids_v[pl.ds(v * L, L)]
            off = idv - rbase
            m = (off >= 0) & (off < RNG)
            offs = jnp.where(m, off, 0)
            val = v * L + iota
            plsc.store_scatter(win_t, [offs], val, mask=m)
            offs_l.append(offs)
            m_l.append(m)
            val_l.append(val)
        bad = None
        for t in range(G):
            got = plsc.load_gather(win_t, [offs_l[t]])
            bt = m_l[t] & (got != val_l[t])
            bad = bt if bad is None else (bad | bt)

        @pl.when(jnp.any(bad))
        def _fix():
            for t in range(G):
                for lane in range(L):
                    plsc.store_scatter(win_t, [offs_l[t]], val_l[t],
                                       mask=m_l[t] & (iota == lane))

        return carry

    lax.fori_loop(0, NVEC // G, wbody, 0)


@functools.partial(
    pl.kernel,
    out_type=(
        _f32((B,)),
        _f32((B,)),
        _f32((B,)),
        _f32((NW, L)),
        _f32((NW, L)),
    ),
    mesh=_mesh,
    compiler_params=pltpu.CompilerParams(needs_layout_passes=False),
    scratch_types=(
        pltpu.VMEM((4, 128), jnp.int32),
        pltpu.VMEM((4, 128), jnp.int32),
        pltpu.VMEM((SPW,), jnp.float32),
        pltpu.VMEM((SPW,), jnp.float32),
        pltpu.VMEM((SPW,), jnp.float32),
        pltpu.VMEM((SPW,), jnp.float32),
        pltpu.VMEM((SPW,), jnp.float32),
        pltpu.VMEM((SPW,), jnp.float32),
        pltpu.VMEM((SPW,), jnp.float32),
        pltpu.VMEM((SPW,), jnp.float32),
        pltpu.VMEM((SPW,), jnp.float32),
        pltpu.VMEM((SPW,), jnp.float32),
        pltpu.VMEM((SPW,), jnp.float32),
        pltpu.VMEM((SPW,), jnp.float32),
        pltpu.VMEM((L,), jnp.float32),
        pltpu.VMEM((L,), jnp.float32),
        pltpu.SemaphoreType.DMA,
    ),
)
def _phase_b(ids, evph, losh, updx, updy, updz, winh, sval, sw,
             wout, k2o, k2wo, Sp, Swp,
             ids2d, win2d, sle, sll, kx, ky, kz, ew, lw, gv, gw,
             bw, bk2, bk2w, pS, pSw, sem):
    w_ = _wid()
    base = w_ * SPW
    cps = [pltpu.async_copy(ids.at[pl.ds(base + c * 128, 128)], ids2d.at[c],
                            sem) for c in range(4)]
    cps.append(pltpu.async_copy(evph.at[pl.ds(base, SPW)], sle, sem))
    cps.append(pltpu.async_copy(losh.at[pl.ds(base, SPW)], sll, sem))
    for cp in cps:
        cp.wait()
    cps = [pltpu.async_copy(winh.at[ids2d.at[c]], win2d.at[c], sem)
           for c in range(4)]
    for cp in cps:
        cp.wait()
    cps = []
    for c in range(4):
        iw = win2d.at[c]
        ii = ids2d.at[c]
        d = pl.ds(c * 128, 128)
        cps.append(pltpu.async_copy(updx.at[iw], kx.at[d], sem))
        cps.append(pltpu.async_copy(updy.at[iw], ky.at[d], sem))
        cps.append(pltpu.async_copy(updz.at[iw], kz.at[d], sem))
        cps.append(pltpu.async_copy(evph.at[iw], ew.at[d], sem))
        cps.append(pltpu.async_copy(losh.at[iw], lw.at[d], sem))
        cps.append(pltpu.async_copy(sval.at[ii], gv.at[d], sem))
        cps.append(pltpu.async_copy(sw.at[ii], gw.at[d], sem))
    for cp in cps:
        cp.wait()

    def cbody(j, carry):
        sS, sSw = carry
        s = pl.ds(j * L, L)
        kprod = kx[s] * ky[s] * kz[s]
        wv = sle[s] / kprod
        wwin = ew[s] / kprod
        gws = gw[s]
        nww = D1 * wwin + D9 * gws
        nvv = (D1 * wwin * lw[s] + D9 * gws * gv[s]) / nww
        bw[s] = wv
        bk2[s] = nvv
        bk2w[s] = nww
        return sS + (sll[s] - nvv) * wv, sSw + wv

    zero = jnp.zeros((L,), jnp.float32)
    sS, sSw = lax.fori_loop(0, SPW // L, cbody, (zero, zero))
    pS[...] = sS
    pSw[...] = sSw
    pltpu.sync_copy(bw, wout.at[pl.ds(base, SPW)])
    pltpu.sync_copy(bk2, k2o.at[pl.ds(base, SPW)])
    pltpu.sync_copy(bk2w, k2wo.at[pl.ds(base, SPW)])
    pltpu.sync_copy(pS, Sp.at[w_])
    pltpu.sync_copy(pSw, Swp.at[w_])


@functools.partial(
    pl.kernel,
    out_type=(_f32((M,)), _f32((M,))),
    mesh=_mesh,
    compiler_params=pltpu.CompilerParams(needs_layout_passes=False),
    scratch_types=(
        pltpu.VMEM((B,), jnp.int32),
        pltpu.VMEM((B,), jnp.float32),
        pltpu.VMEM((B,), jnp.float32),
        pltpu.VMEM((RNG,), jnp.float32),
        pltpu.VMEM((RNG,), jnp.float32),
        pltpu.SemaphoreType.DMA,
    ),
)
def _phase_c(ids, k2, k2w, sval, sw,
             oval, ow,
             ids_v, k2v, k2wv, segv, segw, sem):
    w_ = _wid()
    rbase = w_ * RNG
    last = NW - 1
    cps = [pltpu.async_copy(ids, ids_v, sem),
           pltpu.async_copy(k2, k2v, sem),
           pltpu.async_copy(k2w, k2wv, sem)]

    @pl.when(w_ < last)
    def _load_full():
        a = pltpu.async_copy(sval.at[pl.ds(rbase, RNG)], segv, sem)
        b = pltpu.async_copy(sw.at[pl.ds(rbase, RNG)], segw, sem)
        a.wait()
        b.wait()

    @pl.when(w_ == last)
    def _load_tail():
        a = pltpu.async_copy(sval.at[pl.ds(rbase, TAIL)],
                             segv.at[pl.ds(0, TAIL)], sem)
        b = pltpu.async_copy(sw.at[pl.ds(rbase, TAIL)],
                             segw.at[pl.ds(0, TAIL)], sem)
        a.wait()
        b.wait()

    for cp in cps:
        cp.wait()

    def abody(v, carry):
        s = pl.ds(v * L, L)
        idv = ids_v[s]
        off = idv - rbase
        m = (off >= 0) & (off < RNG)
        offs = jnp.where(m, off, 0)
        plsc.store_scatter(segv, [offs], k2v[s], mask=m)
        plsc.store_scatter(segw, [offs], k2wv[s], mask=m)
        return carry

    lax.fori_loop(0, NVEC, abody, 0, unroll=8)

    @pl.when(w_ < last)
    def _store_full():
        a = pltpu.async_copy(segv, oval.at[pl.ds(rbase, RNG)], sem)
        b = pltpu.async_copy(segw, ow.at[pl.ds(rbase, RNG)], sem)
        a.wait()
        b.wait()

    @pl.when(w_ == last)
    def _store_tail():
        a = pltpu.async_copy(segv.at[pl.ds(0, TAIL)],
                             oval.at[pl.ds(rbase, TAIL)], sem)
        b = pltpu.async_copy(segw.at[pl.ds(0, TAIL)],
                             ow.at[pl.ds(rbase, TAIL)], sem)
        a.wait()
        b.wait()


def kernel(dist_a, dist_b, x, y, z, target, ids, xstore, ystore, zstore,
           store_val, store_w):
    win, updx, updy, updz, evp, los = _phase_a(
        ids, x, y, z, dist_a, dist_b, target, xstore, ystore, zstore)
    wout, k2, k2w, Sp, Swp = _phase_b(
        ids, evp, los, updx, updy, updz, win, store_val, store_w)
    oval, ow = _phase_c(ids, k2, k2w, store_val, store_w)
    n = (Swp.sum() + 1e-5) / B
    final = Sp.sum() / n
    return final, wout, oval, ow
